# baseline (speedup 1.0000x reference)
import functools
import math

import jax
import jax.numpy as jnp
from jax import lax
from jax.experimental import pallas as pl
from jax.experimental.pallas import tpu as pltpu
from jax.experimental.pallas import tpu_sc as plsc

_N = 10000
_E = 320000
_D = 128
_NG = 50
_NF = 64
_CUTOFF = 10.0
_LN2 = math.log(2.0)

_NC, _NS, _L = 2, 16, 16
_NW = _NC * _NS
_E_PAD = 327680
_EPT = _E_PAD // _NW

_CHA = 512
_CHB = 128
_NP = 10240
_RPS = _NP // _NS
_ZR = 128

_MESH = dict(core_axis_name="c", subcore_axis_name="s")


def _ssp(v):
    return jnp.maximum(v, 0.0) + jnp.log(1.0 + jnp.exp(-jnp.abs(v))) - _LN2


@functools.partial(
    pl.kernel,
    out_type=jax.ShapeDtypeStruct((_E_PAD,), jnp.float32),
    mesh=plsc.VectorSubcoreMesh(**_MESH),
    compiler_params=pltpu.CompilerParams(needs_layout_passes=False, use_tc_tiling_on_sc=False),
    scratch_types=[
        pltpu.VMEM((_N,), jnp.float32),
        pltpu.VMEM((_N,), jnp.float32),
        pltpu.VMEM((_N,), jnp.float32),
        pltpu.VMEM((_CHA,), jnp.int32),
        pltpu.VMEM((_CHA,), jnp.int32),
        pltpu.VMEM((_CHA,), jnp.float32),
    ],
)
def _sc_dist(px_hbm, py_hbm, pz_hbm, ei_hbm, d2_hbm,
             px, py, pz, sv, dv, ov):
    c = lax.axis_index("c")
    s = lax.axis_index("s")
    wid = s * _NC + c
    pltpu.sync_copy(px_hbm, px)
    pltpu.sync_copy(py_hbm, py)
    pltpu.sync_copy(pz_hbm, pz)
    nk = jnp.maximum(jnp.minimum(_E, (wid + 1) * _EPT) - wid * _EPT, 0) // _CHA

    def chunk(k, carry):
        base = wid * _EPT + k * _CHA
        pltpu.sync_copy(ei_hbm.at[0, pl.ds(base, _CHA)], sv)
        pltpu.sync_copy(ei_hbm.at[1, pl.ds(base, _CHA)], dv)

        def inner(i, carry2):
            s16 = sv[pl.ds(i * _L, _L)]
            d16 = dv[pl.ds(i * _L, _L)]
            dx = plsc.load_gather(px, [d16]) - plsc.load_gather(px, [s16])
            dy = plsc.load_gather(py, [d16]) - plsc.load_gather(py, [s16])
            dz = plsc.load_gather(pz, [d16]) - plsc.load_gather(pz, [s16])
            ov[pl.ds(i * _L, _L)] = dx * dx + dy * dy + dz * dz
            return carry2

        lax.fori_loop(0, _CHA // _L, inner, 0)
        pltpu.sync_copy(ov, d2_hbm.at[pl.ds(base, _CHA)])
        return carry

    lax.fori_loop(0, nk, chunk, 0)


@functools.partial(
    pl.kernel,
    out_type=jax.ShapeDtypeStruct((_NC, _NP, _NF), jnp.float32),
    mesh=plsc.VectorSubcoreMesh(**_MESH),
    compiler_params=pltpu.CompilerParams(needs_layout_passes=False, use_tc_tiling_on_sc=False),
    scratch_types=[
        pltpu.VMEM_SHARED((_NP, _NF), jnp.float32),
        pltpu.VMEM_SHARED((_NP, _NF), jnp.float32),
        pltpu.VMEM((_CHB,), jnp.int32),
        pltpu.VMEM((_CHB,), jnp.int32),
        pltpu.VMEM((_CHB, _NF), jnp.float32),
        pltpu.VMEM((_CHB // 2, 2 * _NF), jnp.float32),
        pltpu.VMEM((_ZR, _NF), jnp.float32),
    ],
)
def _sc_msg(h_hbm, w_hbm, ei_hbm, out_hbm,
            acc, hsh, sv, dv, hs, wv, zb):
    c = lax.axis_index("c")
    s = lax.axis_index("s")
    wid = s * _NC + c
    z16 = jnp.zeros((_L,), jnp.float32)

    def zrow(r, carry):
        for ci in range(_NF // _L):
            zb[r, pl.ds(ci * _L, _L)] = z16
        return carry

    lax.fori_loop(0, _ZR, zrow, 0)
    row_s = s * _RPS
    for k2 in range(_RPS // _ZR):
        pltpu.sync_copy(zb, acc.at[pl.ds(row_s + k2 * _ZR, _ZR)])
    pltpu.sync_copy(h_hbm.at[pl.ds(row_s, _RPS)], hsh.at[pl.ds(row_s, _RPS)])
    plsc.subcore_barrier()

    def chunk(k, carry):
        base = wid * _EPT + k * _CHB
        pltpu.sync_copy(ei_hbm.at[0, pl.ds(base, _CHB)], sv)
        pltpu.sync_copy(ei_hbm.at[1, pl.ds(base, _CHB)], dv)
        pltpu.sync_copy(hsh.at[sv], hs)
        pltpu.sync_copy(w_hbm.at[pl.ds(base // 2, _CHB // 2)], wv)

        def mul(j, carry2):
            for ci in range(_NF // _L):
                sl = pl.ds(ci * _L, _L)
                hs[2 * j, sl] = hs[2 * j, sl] * wv[j, pl.ds(ci * _L, _L)]
                hs[2 * j + 1, sl] = hs[2 * j + 1, sl] * wv[j, pl.ds(_NF + ci * _L, _L)]
            return carry2

        lax.fori_loop(0, _CHB // 2, mul, 0)
        pltpu.sync_copy(hs, acc.at[dv], add=True)
        return carry

    lax.fori_loop(0, _EPT // _CHB, chunk, 0)
    plsc.subcore_barrier()

    for k2 in range(_RPS // _ZR):
        row0 = row_s + k2 * _ZR
        pltpu.sync_copy(acc.at[pl.ds(row0, _ZR)], zb)
        pltpu.sync_copy(zb, out_hbm.at[c, pl.ds(row0, _ZR)])


def _h_body(x_ref, w_ref, o_ref):
    o_ref[...] = jnp.dot(x_ref[...], w_ref[...],
                         preferred_element_type=jnp.float32)


def _tc_h(x, w_cf1):
    return pl.pallas_call(
        _h_body,
        grid=(10,),
        in_specs=[
            pl.BlockSpec((_N // 10, _D), lambda i: (i, 0)),
            pl.BlockSpec((_D, _NF), lambda i: (0, 0)),
        ],
        out_specs=pl.BlockSpec((_N // 10, _NF), lambda i: (i, 0)),
        out_shape=jax.ShapeDtypeStruct((_N, _NF), jnp.float32),
    )(x, w_cf1)


_BE = 2048


def _filter_body(d2_ref, w1t_ref, b1_ref, w2t_ref, b2_ref, o_ref):
    pid = pl.program_id(0)
    d2 = d2_ref[...].reshape(1, _BE)
    dist = jnp.sqrt(d2 + 1e-12)
    cw = 0.5 * (jnp.cos(dist * (math.pi / _CUTOFF)) + 1.0)
    gi = lax.broadcasted_iota(jnp.int32, (_NG, _BE), 0)
    g = gi.astype(jnp.float32)
    delta = _CUTOFF / (_NG - 1)
    coeff = -0.5 / (delta * delta)
    diff = dist - g * delta
    ea = jnp.exp(coeff * diff * diff)
    t = jnp.dot(w1t_ref[...], ea, preferred_element_type=jnp.float32)
    t = _ssp(t + b1_ref[...])
    t = jnp.dot(w2t_ref[...], t, preferred_element_type=jnp.float32)
    t = t + b2_ref[...]
    eid = pid * _BE + lax.broadcasted_iota(jnp.int32, (1, _BE), 1)
    t = t * jnp.where(eid < _E, cw, 0.0)
    ii = (lax.broadcasted_iota(jnp.int32, (_NF, _NF), 0)
          == lax.broadcasted_iota(jnp.int32, (_NF, _NF), 1)).astype(jnp.float32)
    dn = (((0,), (0,)), ((), ()))
    ta = lax.dot_general(t[:, : _BE // 2], ii, dn,
                         preferred_element_type=jnp.float32)
    tb = lax.dot_general(t[:, _BE // 2:], ii, dn,
                         preferred_element_type=jnp.float32)
    o_ref[...] = jnp.concatenate([ta, tb], axis=1)


def _tc_filter(d2, w1t, b1, w2t, b2):
    return pl.pallas_call(
        _filter_body,
        grid=(_E_PAD // _BE,),
        in_specs=[
            pl.BlockSpec((_BE,), lambda i: (i,)),
            pl.BlockSpec((_NF, _NG), lambda i: (0, 0)),
            pl.BlockSpec((_NF, 1), lambda i: (0, 0)),
            pl.BlockSpec((_NF, _NF), lambda i: (0, 0)),
            pl.BlockSpec((_NF, 1), lambda i: (0, 0)),
        ],
        out_specs=pl.BlockSpec((_BE // 2, 2 * _NF), lambda i: (i, 0)),
        out_shape=jax.ShapeDtypeStruct((_E_PAD // 2, 2 * _NF), jnp.float32),
    )(d2, w1t, b1, w2t, b2)


def _tail_body(x_ref, p0_ref, p1_ref, wcf2_ref, bcf2_ref, wint_ref, bint_ref,
               wlin_ref, blin_ref, o_ref):
    agg = p0_ref[0] + p1_ref[0]
    h2 = jnp.dot(agg, wcf2_ref[...], preferred_element_type=jnp.float32)
    h2 = _ssp(h2 + bcf2_ref[...])
    h2 = jnp.dot(h2, wint_ref[...], preferred_element_type=jnp.float32)
    h2 = h2 + bint_ref[...]
    y = jnp.dot(h2, wlin_ref[...], preferred_element_type=jnp.float32)
    y = jnp.maximum(y + blin_ref[...], 0.0)
    o_ref[...] = x_ref[...] + y


def _tc_tail(x, parts, w_cf2, b_cf2, w_int, b_int, w_lin1, b_lin1):
    br = _N // 10
    return pl.pallas_call(
        _tail_body,
        grid=(10,),
        in_specs=[
            pl.BlockSpec((br, _D), lambda i: (i, 0)),
            pl.BlockSpec((1, br, _NF), lambda i: (0, i, 0)),
            pl.BlockSpec((1, br, _NF), lambda i: (1, i, 0)),
            pl.BlockSpec((_NF, _D), lambda i: (0, 0)),
            pl.BlockSpec((1, _D), lambda i: (0, 0)),
            pl.BlockSpec((_D, _D), lambda i: (0, 0)),
            pl.BlockSpec((1, _D), lambda i: (0, 0)),
            pl.BlockSpec((_D, _D), lambda i: (0, 0)),
            pl.BlockSpec((1, _D), lambda i: (0, 0)),
        ],
        out_specs=pl.BlockSpec((br, _D), lambda i: (i, 0)),
        out_shape=jax.ShapeDtypeStruct((_N, _D), jnp.float32),
    )(x, parts, parts, w_cf2, b_cf2, w_int, b_int, w_lin1, b_lin1)


def kernel(x, pos, edge_index, W_mlp1, b_mlp1, W_mlp2, b_mlp2,
           W_cf1, W_cf2, b_cf2, W_int, b_int, W_lin1, b_lin1):
    src = edge_index[0]
    dst = edge_index[1]
    pad = _E_PAD - _E
    srcp = jnp.pad(src, (0, pad))
    dstp = jnp.pad(dst, (0, pad))
    eip = jnp.stack([srcp, dstp])
    posx = pos[:, 0]
    posy = pos[:, 1]
    posz = pos[:, 2]

    d2 = _sc_dist(posx, posy, posz, eip)
    h = _tc_h(x, W_cf1)

    w_edges = _tc_filter(d2, W_mlp1.T, b_mlp1.reshape(_NF, 1),
                         W_mlp2.T, b_mlp2.reshape(_NF, 1))

    sp = srcp.reshape(-1, 2, _BE // 2).transpose(0, 2, 1).reshape(-1)
    dp = dstp.reshape(-1, 2, _BE // 2).transpose(0, 2, 1).reshape(-1)
    hp = jnp.pad(h, ((0, _NP - _N), (0, 0)))
    parts = _sc_msg(hp, w_edges, jnp.stack([sp, dp]))

    return _tc_tail(x, parts, W_cf2, b_cf2.reshape(1, _D),
                    W_int, b_int.reshape(1, _D), W_lin1, b_lin1.reshape(1, _D))


# speedup vs baseline: 1.4268x; 1.4268x over previous
import functools
import math

import jax
import jax.numpy as jnp
from jax import lax
from jax.experimental import pallas as pl
from jax.experimental.pallas import tpu as pltpu
from jax.experimental.pallas import tpu_sc as plsc

_N = 10000
_E = 320000
_D = 128
_NG = 50
_NF = 64
_CUTOFF = 10.0
_LN2 = math.log(2.0)

_NC, _NS, _L = 2, 16, 16
_NW = _NC * _NS
_E_PAD = 327680
_EPT = _E_PAD // _NW

_CHA = 512
_CHB = 128
_NP = 10240
_RPS = _NP // _NS
_ZR = 128

_MESH = dict(core_axis_name="c", subcore_axis_name="s")


def _ssp(v):
    return jnp.maximum(v, 0.0) + jnp.log(1.0 + jnp.exp(-jnp.abs(v))) - _LN2


@functools.partial(
    pl.kernel,
    out_type=jax.ShapeDtypeStruct((_E_PAD,), jnp.float32),
    mesh=plsc.VectorSubcoreMesh(**_MESH),
    compiler_params=pltpu.CompilerParams(needs_layout_passes=False, use_tc_tiling_on_sc=False),
    scratch_types=[
        pltpu.VMEM((_N,), jnp.float32),
        pltpu.VMEM((_N,), jnp.float32),
        pltpu.VMEM((_N,), jnp.float32),
        pltpu.VMEM((_CHA,), jnp.int32),
        pltpu.VMEM((_CHA,), jnp.int32),
        pltpu.VMEM((_CHA,), jnp.float32),
    ],
)
def _sc_dist(px_hbm, py_hbm, pz_hbm, ei_hbm, d2_hbm,
             px, py, pz, sv, dv, ov):
    c = lax.axis_index("c")
    s = lax.axis_index("s")
    wid = s * _NC + c
    pltpu.sync_copy(px_hbm, px)
    pltpu.sync_copy(py_hbm, py)
    pltpu.sync_copy(pz_hbm, pz)
    nk = jnp.maximum(jnp.minimum(_E, (wid + 1) * _EPT) - wid * _EPT, 0) // _CHA

    def chunk(k, carry):
        base = wid * _EPT + k * _CHA
        pltpu.sync_copy(ei_hbm.at[0, pl.ds(base, _CHA)], sv)
        pltpu.sync_copy(ei_hbm.at[1, pl.ds(base, _CHA)], dv)

        def inner(i, carry2):
            s16 = sv[pl.ds(i * _L, _L)]
            d16 = dv[pl.ds(i * _L, _L)]
            dx = plsc.load_gather(px, [d16]) - plsc.load_gather(px, [s16])
            dy = plsc.load_gather(py, [d16]) - plsc.load_gather(py, [s16])
            dz = plsc.load_gather(pz, [d16]) - plsc.load_gather(pz, [s16])
            ov[pl.ds(i * _L, _L)] = dx * dx + dy * dy + dz * dz
            return carry2

        lax.fori_loop(0, _CHA // _L, inner, 0)
        pltpu.sync_copy(ov, d2_hbm.at[pl.ds(base, _CHA)])
        return carry

    lax.fori_loop(0, nk, chunk, 0)


@functools.partial(
    pl.kernel,
    out_type=jax.ShapeDtypeStruct((_NC, _NP, _NF), jnp.float32),
    mesh=plsc.VectorSubcoreMesh(**_MESH),
    compiler_params=pltpu.CompilerParams(needs_layout_passes=False, use_tc_tiling_on_sc=False),
    scratch_types=[
        pltpu.VMEM_SHARED((_NP, _NF), jnp.float32),
        pltpu.VMEM_SHARED((_NP, _NF), jnp.float32),
        pltpu.VMEM((2, _CHB), jnp.int32),
        pltpu.VMEM((2, _CHB), jnp.int32),
        pltpu.VMEM((_CHB, _NF), jnp.float32),
        pltpu.VMEM((2, _CHB // 2, 2 * _NF), jnp.float32),
        pltpu.VMEM((_ZR, _NF), jnp.float32),
        pltpu.SemaphoreType.DMA,
        pltpu.SemaphoreType.DMA,
        pltpu.SemaphoreType.DMA,
        pltpu.SemaphoreType.DMA,
        pltpu.SemaphoreType.DMA,
        pltpu.SemaphoreType.DMA,
    ],
)
def _sc_msg(h_hbm, w_hbm, src_hbm, dst_hbm, out_hbm,
            acc, hsh, sv, dv, hs, wv, zb, ss0, ss1, sd0, sd1, sw0, sw1):
    c = lax.axis_index("c")
    s = lax.axis_index("s")
    wid = s * _NC + c
    z16 = jnp.zeros((_L,), jnp.float32)
    ss = (ss0, ss1)
    sd = (sd0, sd1)
    sw = (sw0, sw1)
    nk = _EPT // _CHB

    def zrow(r, carry):
        for ci in range(_NF // _L):
            zb[r, pl.ds(ci * _L, _L)] = z16
        return carry

    lax.fori_loop(0, _ZR, zrow, 0)
    row_s = s * _RPS
    for k2 in range(_RPS // _ZR):
        pltpu.sync_copy(zb, acc.at[pl.ds(row_s + k2 * _ZR, _ZR)])
    pltpu.sync_copy(h_hbm.at[pl.ds(row_s, _RPS)], hsh.at[pl.ds(row_s, _RPS)])
    plsc.subcore_barrier()

    def start(slot, k):
        base = wid * _EPT + k * _CHB
        pltpu.async_copy(src_hbm.at[pl.ds(base, _CHB)], sv.at[slot], ss[slot])
        pltpu.async_copy(dst_hbm.at[pl.ds(base, _CHB)], dv.at[slot], sd[slot])
        pltpu.async_copy(w_hbm.at[pl.ds(base // 2, _CHB // 2)], wv.at[slot],
                         sw[slot])

    def wait_idx(slot):
        pltpu.make_async_copy(src_hbm.at[pl.ds(0, _CHB)], sv.at[slot],
                              ss[slot]).wait()
        pltpu.make_async_copy(dst_hbm.at[pl.ds(0, _CHB)], dv.at[slot],
                              sd[slot]).wait()
        pltpu.make_async_copy(w_hbm.at[pl.ds(0, _CHB // 2)], wv.at[slot],
                              sw[slot]).wait()

    def process(slot):
        pltpu.sync_copy(hsh.at[sv.at[slot]], hs)

        def mul(j, carry2):
            for ci in range(_NF // _L):
                sl = pl.ds(ci * _L, _L)
                hs[2 * j, sl] = hs[2 * j, sl] * wv[slot, j, pl.ds(ci * _L, _L)]
                hs[2 * j + 1, sl] = (hs[2 * j + 1, sl]
                                     * wv[slot, j, pl.ds(_NF + ci * _L, _L)])
            return carry2

        lax.fori_loop(0, _CHB // 2, mul, 0)
        pltpu.sync_copy(hs, acc.at[dv.at[slot]], add=True)

    start(0, 0)

    def pair(kk, carry):
        k0 = 2 * kk
        start(1, k0 + 1)
        wait_idx(0)
        process(0)
        start(0, lax.rem(k0 + 2, nk))
        wait_idx(1)
        process(1)
        return carry

    lax.fori_loop(0, nk // 2, pair, 0)
    wait_idx(0)
    plsc.subcore_barrier()

    for k2 in range(_RPS // _ZR):
        row0 = row_s + k2 * _ZR
        pltpu.sync_copy(acc.at[pl.ds(row0, _ZR)], zb)
        pltpu.sync_copy(zb, out_hbm.at[c, pl.ds(row0, _ZR)])


def _h_body(x_ref, w_ref, o_ref):
    o_ref[...] = jnp.dot(x_ref[...], w_ref[...],
                         preferred_element_type=jnp.float32)


def _tc_h(x, w_cf1):
    return pl.pallas_call(
        _h_body,
        grid=(10,),
        in_specs=[
            pl.BlockSpec((_N // 10, _D), lambda i: (i, 0)),
            pl.BlockSpec((_D, _NF), lambda i: (0, 0)),
        ],
        out_specs=pl.BlockSpec((_N // 10, _NF), lambda i: (i, 0)),
        out_shape=jax.ShapeDtypeStruct((_N, _NF), jnp.float32),
    )(x, w_cf1)


_BE = 2048


def _filter_body(d2_ref, w1t_ref, b1_ref, w2t_ref, b2_ref, o_ref):
    pid = pl.program_id(0)
    d2 = d2_ref[...].reshape(1, _BE)
    dist = jnp.sqrt(d2 + 1e-12)
    cw = 0.5 * (jnp.cos(dist * (math.pi / _CUTOFF)) + 1.0)
    gi = lax.broadcasted_iota(jnp.int32, (_NG, _BE), 0)
    g = gi.astype(jnp.float32)
    delta = _CUTOFF / (_NG - 1)
    coeff = -0.5 / (delta * delta)
    diff = dist - g * delta
    ea = jnp.exp(coeff * diff * diff)
    t = jnp.dot(w1t_ref[...], ea, preferred_element_type=jnp.float32)
    t = _ssp(t + b1_ref[...])
    t = jnp.dot(w2t_ref[...], t, preferred_element_type=jnp.float32)
    t = t + b2_ref[...]
    eid = pid * _BE + lax.broadcasted_iota(jnp.int32, (1, _BE), 1)
    t = t * jnp.where(eid < _E, cw, 0.0)
    ii = (lax.broadcasted_iota(jnp.int32, (_NF, _NF), 0)
          == lax.broadcasted_iota(jnp.int32, (_NF, _NF), 1)).astype(jnp.float32)
    dn = (((0,), (0,)), ((), ()))
    ta = lax.dot_general(t[:, : _BE // 2], ii, dn,
                         preferred_element_type=jnp.float32)
    tb = lax.dot_general(t[:, _BE // 2:], ii, dn,
                         preferred_element_type=jnp.float32)
    o_ref[...] = jnp.concatenate([ta, tb], axis=1)


def _tc_filter(d2, w1t, b1, w2t, b2):
    return pl.pallas_call(
        _filter_body,
        grid=(_E_PAD // _BE,),
        in_specs=[
            pl.BlockSpec((_BE,), lambda i: (i,)),
            pl.BlockSpec((_NF, _NG), lambda i: (0, 0)),
            pl.BlockSpec((_NF, 1), lambda i: (0, 0)),
            pl.BlockSpec((_NF, _NF), lambda i: (0, 0)),
            pl.BlockSpec((_NF, 1), lambda i: (0, 0)),
        ],
        out_specs=pl.BlockSpec((_BE // 2, 2 * _NF), lambda i: (i, 0)),
        out_shape=jax.ShapeDtypeStruct((_E_PAD // 2, 2 * _NF), jnp.float32),
    )(d2, w1t, b1, w2t, b2)


def _tail_body(x_ref, p0_ref, p1_ref, wcf2_ref, bcf2_ref, wint_ref, bint_ref,
               wlin_ref, blin_ref, o_ref):
    agg = p0_ref[0] + p1_ref[0]
    h2 = jnp.dot(agg, wcf2_ref[...], preferred_element_type=jnp.float32)
    h2 = _ssp(h2 + bcf2_ref[...])
    h2 = jnp.dot(h2, wint_ref[...], preferred_element_type=jnp.float32)
    h2 = h2 + bint_ref[...]
    y = jnp.dot(h2, wlin_ref[...], preferred_element_type=jnp.float32)
    y = jnp.maximum(y + blin_ref[...], 0.0)
    o_ref[...] = x_ref[...] + y


def _tc_tail(x, parts, w_cf2, b_cf2, w_int, b_int, w_lin1, b_lin1):
    br = _N // 10
    return pl.pallas_call(
        _tail_body,
        grid=(10,),
        in_specs=[
            pl.BlockSpec((br, _D), lambda i: (i, 0)),
            pl.BlockSpec((1, br, _NF), lambda i: (0, i, 0)),
            pl.BlockSpec((1, br, _NF), lambda i: (1, i, 0)),
            pl.BlockSpec((_NF, _D), lambda i: (0, 0)),
            pl.BlockSpec((1, _D), lambda i: (0, 0)),
            pl.BlockSpec((_D, _D), lambda i: (0, 0)),
            pl.BlockSpec((1, _D), lambda i: (0, 0)),
            pl.BlockSpec((_D, _D), lambda i: (0, 0)),
            pl.BlockSpec((1, _D), lambda i: (0, 0)),
        ],
        out_specs=pl.BlockSpec((br, _D), lambda i: (i, 0)),
        out_shape=jax.ShapeDtypeStruct((_N, _D), jnp.float32),
    )(x, parts, parts, w_cf2, b_cf2, w_int, b_int, w_lin1, b_lin1)


def kernel(x, pos, edge_index, W_mlp1, b_mlp1, W_mlp2, b_mlp2,
           W_cf1, W_cf2, b_cf2, W_int, b_int, W_lin1, b_lin1):
    src = edge_index[0]
    dst = edge_index[1]
    pad = _E_PAD - _E
    srcp = jnp.pad(src, (0, pad))
    dstp = jnp.pad(dst, (0, pad))
    eip = jnp.stack([srcp, dstp])
    posx = pos[:, 0]
    posy = pos[:, 1]
    posz = pos[:, 2]

    d2 = _sc_dist(posx, posy, posz, eip)
    h = _tc_h(x, W_cf1)

    w_edges = _tc_filter(d2, W_mlp1.T, b_mlp1.reshape(_NF, 1),
                         W_mlp2.T, b_mlp2.reshape(_NF, 1))

    sp = srcp.reshape(-1, 2, _BE // 2).transpose(0, 2, 1).reshape(-1)
    dp = dstp.reshape(-1, 2, _BE // 2).transpose(0, 2, 1).reshape(-1)
    hp = jnp.pad(h, ((0, _NP - _N), (0, 0)))
    parts = _sc_msg(hp, w_edges, sp, dp)

    return _tc_tail(x, parts, W_cf2, b_cf2.reshape(1, _D),
                    W_int, b_int.reshape(1, _D), W_lin1, b_lin1.reshape(1, _D))


# speedup vs baseline: 1.4833x; 1.0396x over previous
import functools
import math

import jax
import jax.numpy as jnp
from jax import lax
from jax.experimental import pallas as pl
from jax.experimental.pallas import tpu as pltpu
from jax.experimental.pallas import tpu_sc as plsc

_N = 10000
_E = 320000
_D = 128
_NG = 50
_NF = 64
_CUTOFF = 10.0
_LN2 = math.log(2.0)

_NC, _NS, _L = 2, 16, 16
_NW = _NC * _NS
_E_PAD = 327680
_EPT = _E_PAD // _NW

_EH = _E_PAD // 2
_EPTH = _EH // _NW
_EVA = _EH
_EVB = _E - _EH

_CHA = 512
_CHB = 128
_NP = 10240
_RPS = _NP // _NS
_ZR = 128

_MESH = dict(core_axis_name="c", subcore_axis_name="s")


def _ssp(v):
    return jnp.maximum(v, 0.0) + jnp.log(1.0 + jnp.exp(-jnp.abs(v))) - _LN2


def _make_sc_dist(ev):
    @functools.partial(
        pl.kernel,
        out_type=jax.ShapeDtypeStruct((_EH,), jnp.float32),
        mesh=plsc.VectorSubcoreMesh(**_MESH),
        compiler_params=pltpu.CompilerParams(needs_layout_passes=False, use_tc_tiling_on_sc=False),
        scratch_types=[
            pltpu.VMEM((_N,), jnp.float32),
            pltpu.VMEM((_N,), jnp.float32),
            pltpu.VMEM((_N,), jnp.float32),
            pltpu.VMEM((_CHA,), jnp.int32),
            pltpu.VMEM((_CHA,), jnp.int32),
            pltpu.VMEM((_CHA,), jnp.float32),
        ],
    )
    def _dist(px_hbm, py_hbm, pz_hbm, ei_hbm, d2_hbm,
              px, py, pz, sv, dv, ov):
        c = lax.axis_index("c")
        s = lax.axis_index("s")
        wid = s * _NC + c
        pltpu.sync_copy(px_hbm, px)
        pltpu.sync_copy(py_hbm, py)
        pltpu.sync_copy(pz_hbm, pz)
        nk = (jnp.maximum(jnp.minimum(ev, (wid + 1) * _EPTH) - wid * _EPTH, 0)
              // _CHA)

        def chunk(k, carry):
            base = wid * _EPTH + k * _CHA
            pltpu.sync_copy(ei_hbm.at[0, pl.ds(base, _CHA)], sv)
            pltpu.sync_copy(ei_hbm.at[1, pl.ds(base, _CHA)], dv)

            def inner(i, carry2):
                s16 = sv[pl.ds(i * _L, _L)]
                d16 = dv[pl.ds(i * _L, _L)]
                dx = plsc.load_gather(px, [d16]) - plsc.load_gather(px, [s16])
                dy = plsc.load_gather(py, [d16]) - plsc.load_gather(py, [s16])
                dz = plsc.load_gather(pz, [d16]) - plsc.load_gather(pz, [s16])
                ov[pl.ds(i * _L, _L)] = dx * dx + dy * dy + dz * dz
                return carry2

            lax.fori_loop(0, _CHA // _L, inner, 0)
            pltpu.sync_copy(ov, d2_hbm.at[pl.ds(base, _CHA)])
            return carry

        lax.fori_loop(0, nk, chunk, 0)

    return _dist


_sc_dist_a = _make_sc_dist(_EVA)
_sc_dist_b = _make_sc_dist(_EVB)


@functools.partial(
    pl.kernel,
    out_type=jax.ShapeDtypeStruct((_NC, _NP, _NF), jnp.float32),
    mesh=plsc.VectorSubcoreMesh(**_MESH),
    compiler_params=pltpu.CompilerParams(needs_layout_passes=False, use_tc_tiling_on_sc=False),
    scratch_types=[
        pltpu.VMEM_SHARED((_NP, _NF), jnp.float32),
        pltpu.VMEM_SHARED((_NP, _NF), jnp.float32),
        pltpu.VMEM((2, _CHB), jnp.int32),
        pltpu.VMEM((2, _CHB), jnp.int32),
        pltpu.VMEM((_CHB, _NF), jnp.float32),
        pltpu.VMEM((2, _CHB // 2, 2 * _NF), jnp.float32),
        pltpu.VMEM((_ZR, _NF), jnp.float32),
        pltpu.SemaphoreType.DMA,
        pltpu.SemaphoreType.DMA,
        pltpu.SemaphoreType.DMA,
        pltpu.SemaphoreType.DMA,
        pltpu.SemaphoreType.DMA,
        pltpu.SemaphoreType.DMA,
    ],
)
def _sc_msg(h_hbm, w_hbm, src_hbm, dst_hbm, out_hbm,
            acc, hsh, sv, dv, hs, wv, zb, ss0, ss1, sd0, sd1, sw0, sw1):
    c = lax.axis_index("c")
    s = lax.axis_index("s")
    wid = s * _NC + c
    z16 = jnp.zeros((_L,), jnp.float32)
    ss = (ss0, ss1)
    sd = (sd0, sd1)
    sw = (sw0, sw1)
    nk = _EPTH // _CHB

    def zrow(r, carry):
        for ci in range(_NF // _L):
            zb[r, pl.ds(ci * _L, _L)] = z16
        return carry

    lax.fori_loop(0, _ZR, zrow, 0)
    row_s = s * _RPS
    for k2 in range(_RPS // _ZR):
        pltpu.sync_copy(zb, acc.at[pl.ds(row_s + k2 * _ZR, _ZR)])
    pltpu.sync_copy(h_hbm.at[pl.ds(row_s, _RPS)], hsh.at[pl.ds(row_s, _RPS)])
    plsc.subcore_barrier()

    def start(slot, k):
        base = wid * _EPTH + k * _CHB
        pltpu.async_copy(src_hbm.at[pl.ds(base, _CHB)], sv.at[slot], ss[slot])
        pltpu.async_copy(dst_hbm.at[pl.ds(base, _CHB)], dv.at[slot], sd[slot])
        pltpu.async_copy(w_hbm.at[pl.ds(base // 2, _CHB // 2)], wv.at[slot],
                         sw[slot])

    def wait_idx(slot):
        pltpu.make_async_copy(src_hbm.at[pl.ds(0, _CHB)], sv.at[slot],
                              ss[slot]).wait()
        pltpu.make_async_copy(dst_hbm.at[pl.ds(0, _CHB)], dv.at[slot],
                              sd[slot]).wait()
        pltpu.make_async_copy(w_hbm.at[pl.ds(0, _CHB // 2)], wv.at[slot],
                              sw[slot]).wait()

    def process(slot):
        pltpu.sync_copy(hsh.at[sv.at[slot]], hs)

        def mul(j, carry2):
            for ci in range(_NF // _L):
                sl = pl.ds(ci * _L, _L)
                hs[2 * j, sl] = hs[2 * j, sl] * wv[slot, j, pl.ds(ci * _L, _L)]
                hs[2 * j + 1, sl] = (hs[2 * j + 1, sl]
                                     * wv[slot, j, pl.ds(_NF + ci * _L, _L)])
            return carry2

        lax.fori_loop(0, _CHB // 2, mul, 0)
        pltpu.sync_copy(hs, acc.at[dv.at[slot]], add=True)

    start(0, 0)

    def pair(kk, carry):
        k0 = 2 * kk
        start(1, k0 + 1)
        wait_idx(0)
        process(0)
        start(0, lax.rem(k0 + 2, nk))
        wait_idx(1)
        process(1)
        return carry

    lax.fori_loop(0, nk // 2, pair, 0)
    wait_idx(0)
    plsc.subcore_barrier()

    for k2 in range(_RPS // _ZR):
        row0 = row_s + k2 * _ZR
        pltpu.sync_copy(acc.at[pl.ds(row0, _ZR)], zb)
        pltpu.sync_copy(zb, out_hbm.at[c, pl.ds(row0, _ZR)])


def _h_body(x_ref, w_ref, o_ref):
    o_ref[...] = jnp.dot(x_ref[...], w_ref[...],
                         preferred_element_type=jnp.float32)


def _tc_h(x, w_cf1):
    return pl.pallas_call(
        _h_body,
        grid=(10,),
        in_specs=[
            pl.BlockSpec((_N // 10, _D), lambda i: (i, 0)),
            pl.BlockSpec((_D, _NF), lambda i: (0, 0)),
        ],
        out_specs=pl.BlockSpec((_N // 10, _NF), lambda i: (i, 0)),
        out_shape=jax.ShapeDtypeStruct((_N, _NF), jnp.float32),
    )(x, w_cf1)


_BE = 2048


def _make_filter_body(ev):
    def _filter_body(d2_ref, w1t_ref, b1_ref, w2t_ref, b2_ref, o_ref):
        pid = pl.program_id(0)
        eid = pid * _BE + lax.broadcasted_iota(jnp.int32, (1, _BE), 1)
        valid = eid < ev
        d2 = jnp.where(valid, d2_ref[...].reshape(1, _BE), 0.0)
        dist = jnp.sqrt(d2 + 1e-12)
        cw = 0.5 * (jnp.cos(dist * (math.pi / _CUTOFF)) + 1.0)
        gi = lax.broadcasted_iota(jnp.int32, (_NG, _BE), 0)
        g = gi.astype(jnp.float32)
        delta = _CUTOFF / (_NG - 1)
        coeff = -0.5 / (delta * delta)
        diff = dist - g * delta
        ea = jnp.exp(coeff * diff * diff)
        t = jnp.dot(w1t_ref[...], ea, preferred_element_type=jnp.float32)
        t = _ssp(t + b1_ref[...])
        t = jnp.dot(w2t_ref[...], t, preferred_element_type=jnp.float32)
        t = t + b2_ref[...]
        t = t * jnp.where(valid, cw, 0.0)
        ii = (lax.broadcasted_iota(jnp.int32, (_NF, _NF), 0)
              == lax.broadcasted_iota(jnp.int32, (_NF, _NF), 1)
              ).astype(jnp.float32)
        dn = (((0,), (0,)), ((), ()))
        ta = lax.dot_general(t[:, : _BE // 2], ii, dn,
                             preferred_element_type=jnp.float32)
        tb = lax.dot_general(t[:, _BE // 2:], ii, dn,
                             preferred_element_type=jnp.float32)
        o_ref[...] = jnp.concatenate([ta, tb], axis=1)

    return _filter_body


def _make_tc_filter(ev):
    body = _make_filter_body(ev)

    def _filter(d2, w1t, b1, w2t, b2):
        return pl.pallas_call(
            body,
            grid=(_EH // _BE,),
            in_specs=[
                pl.BlockSpec((_BE,), lambda i: (i,)),
                pl.BlockSpec((_NF, _NG), lambda i: (0, 0)),
                pl.BlockSpec((_NF, 1), lambda i: (0, 0)),
                pl.BlockSpec((_NF, _NF), lambda i: (0, 0)),
                pl.BlockSpec((_NF, 1), lambda i: (0, 0)),
            ],
            out_specs=pl.BlockSpec((_BE // 2, 2 * _NF), lambda i: (i, 0)),
            out_shape=jax.ShapeDtypeStruct((_EH // 2, 2 * _NF), jnp.float32),
        )(d2, w1t, b1, w2t, b2)

    return _filter


_tc_filter_a = _make_tc_filter(_EVA)
_tc_filter_b = _make_tc_filter(_EVB)


def _tail_body(x_ref, p0_ref, p1_ref, p2_ref, p3_ref,
               wcf2_ref, bcf2_ref, wint_ref, bint_ref,
               wlin_ref, blin_ref, o_ref):
    agg = (p0_ref[0] + p1_ref[0]) + (p2_ref[0] + p3_ref[0])
    h2 = jnp.dot(agg, wcf2_ref[...], preferred_element_type=jnp.float32)
    h2 = _ssp(h2 + bcf2_ref[...])
    h2 = jnp.dot(h2, wint_ref[...], preferred_element_type=jnp.float32)
    h2 = h2 + bint_ref[...]
    y = jnp.dot(h2, wlin_ref[...], preferred_element_type=jnp.float32)
    y = jnp.maximum(y + blin_ref[...], 0.0)
    o_ref[...] = x_ref[...] + y


def _tc_tail(x, parts_a, parts_b, w_cf2, b_cf2, w_int, b_int, w_lin1, b_lin1):
    br = _N // 10
    return pl.pallas_call(
        _tail_body,
        grid=(10,),
        in_specs=[
            pl.BlockSpec((br, _D), lambda i: (i, 0)),
            pl.BlockSpec((1, br, _NF), lambda i: (0, i, 0)),
            pl.BlockSpec((1, br, _NF), lambda i: (1, i, 0)),
            pl.BlockSpec((1, br, _NF), lambda i: (0, i, 0)),
            pl.BlockSpec((1, br, _NF), lambda i: (1, i, 0)),
            pl.BlockSpec((_NF, _D), lambda i: (0, 0)),
            pl.BlockSpec((1, _D), lambda i: (0, 0)),
            pl.BlockSpec((_D, _D), lambda i: (0, 0)),
            pl.BlockSpec((1, _D), lambda i: (0, 0)),
            pl.BlockSpec((_D, _D), lambda i: (0, 0)),
            pl.BlockSpec((1, _D), lambda i: (0, 0)),
        ],
        out_specs=pl.BlockSpec((br, _D), lambda i: (i, 0)),
        out_shape=jax.ShapeDtypeStruct((_N, _D), jnp.float32),
    )(x, parts_a, parts_a, parts_b, parts_b,
      w_cf2, b_cf2, w_int, b_int, w_lin1, b_lin1)


def kernel(x, pos, edge_index, W_mlp1, b_mlp1, W_mlp2, b_mlp2,
           W_cf1, W_cf2, b_cf2, W_int, b_int, W_lin1, b_lin1):
    src = edge_index[0]
    dst = edge_index[1]
    pad = _E_PAD - _E
    srcp = jnp.pad(src, (0, pad))
    dstp = jnp.pad(dst, (0, pad))
    eip_a = jnp.stack([srcp[:_EH], dstp[:_EH]])
    eip_b = jnp.stack([srcp[_EH:], dstp[_EH:]])
    posx = pos[:, 0]
    posy = pos[:, 1]
    posz = pos[:, 2]

    d2_a = _sc_dist_a(posx, posy, posz, eip_a)
    d2_b = _sc_dist_b(posx, posy, posz, eip_b)
    h = _tc_h(x, W_cf1)

    w1t = W_mlp1.T
    b1c = b_mlp1.reshape(_NF, 1)
    w2t = W_mlp2.T
    b2c = b_mlp2.reshape(_NF, 1)
    w_a = _tc_filter_a(d2_a, w1t, b1c, w2t, b2c)
    w_b = _tc_filter_b(d2_b, w1t, b1c, w2t, b2c)

    def _ilv(v):
        return v.reshape(-1, 2, _BE // 2).transpose(0, 2, 1).reshape(-1)

    hp = jnp.pad(h, ((0, _NP - _N), (0, 0)))
    parts_a = _sc_msg(hp, w_a, _ilv(srcp[:_EH]), _ilv(dstp[:_EH]))
    parts_b = _sc_msg(hp, w_b, _ilv(srcp[_EH:]), _ilv(dstp[_EH:]))

    return _tc_tail(x, parts_a, parts_b, W_cf2, b_cf2.reshape(1, _D),
                    W_int, b_int.reshape(1, _D), W_lin1, b_lin1.reshape(1, _D))


# speedup vs baseline: 1.4843x; 1.0007x over previous
import functools
import math

import jax
import jax.numpy as jnp
from jax import lax
from jax.experimental import pallas as pl
from jax.experimental.pallas import tpu as pltpu
from jax.experimental.pallas import tpu_sc as plsc

_N = 10000
_E = 320000
_D = 128
_NG = 50
_NF = 64
_CUTOFF = 10.0
_LN2 = math.log(2.0)

_NC, _NS, _L = 2, 16, 16
_NW = _NC * _NS
_E_PAD = 327680
_EPT = _E_PAD // _NW

_EH = _E_PAD // 2
_EPTH = _EH // _NW
_EVA = _EH
_EVB = _E - _EH

_CHA = 512
_CHB = 128
_NP = 10240
_RPS = _NP // _NS
_ZR = 128

_MESH = dict(core_axis_name="c", subcore_axis_name="s")


def _ssp(v):
    return jnp.maximum(v, 0.0) + jnp.log(1.0 + jnp.exp(-jnp.abs(v))) - _LN2


def _make_sc_dist(ev):
    @functools.partial(
        pl.kernel,
        out_type=jax.ShapeDtypeStruct((_E_PAD,), jnp.float32),
        mesh=plsc.VectorSubcoreMesh(**_MESH),
        compiler_params=pltpu.CompilerParams(needs_layout_passes=False, use_tc_tiling_on_sc=False),
        scratch_types=[
            pltpu.VMEM((_N,), jnp.float32),
            pltpu.VMEM((_N,), jnp.float32),
            pltpu.VMEM((_N,), jnp.float32),
            pltpu.VMEM((_CHA,), jnp.int32),
            pltpu.VMEM((_CHA,), jnp.int32),
            pltpu.VMEM((_CHA,), jnp.float32),
        ],
    )
    def _dist(px_hbm, py_hbm, pz_hbm, ei_hbm, d2_hbm,
              px, py, pz, sv, dv, ov):
        c = lax.axis_index("c")
        s = lax.axis_index("s")
        wid = s * _NC + c
        pltpu.sync_copy(px_hbm, px)
        pltpu.sync_copy(py_hbm, py)
        pltpu.sync_copy(pz_hbm, pz)
        nk = (jnp.maximum(jnp.minimum(ev, (wid + 1) * _EPT) - wid * _EPT, 0)
              // _CHA)

        def chunk(k, carry):
            base = wid * _EPT + k * _CHA
            pltpu.sync_copy(ei_hbm.at[0, pl.ds(base, _CHA)], sv)
            pltpu.sync_copy(ei_hbm.at[1, pl.ds(base, _CHA)], dv)

            def inner(i, carry2):
                s16 = sv[pl.ds(i * _L, _L)]
                d16 = dv[pl.ds(i * _L, _L)]
                dx = plsc.load_gather(px, [d16]) - plsc.load_gather(px, [s16])
                dy = plsc.load_gather(py, [d16]) - plsc.load_gather(py, [s16])
                dz = plsc.load_gather(pz, [d16]) - plsc.load_gather(pz, [s16])
                ov[pl.ds(i * _L, _L)] = dx * dx + dy * dy + dz * dz
                return carry2

            lax.fori_loop(0, _CHA // _L, inner, 0)
            pltpu.sync_copy(ov, d2_hbm.at[pl.ds(base, _CHA)])
            return carry

        lax.fori_loop(0, nk, chunk, 0)

    return _dist


_sc_dist = _make_sc_dist(_E)


@functools.partial(
    pl.kernel,
    out_type=jax.ShapeDtypeStruct((_NC, _NP, _NF), jnp.float32),
    mesh=plsc.VectorSubcoreMesh(**_MESH),
    compiler_params=pltpu.CompilerParams(needs_layout_passes=False, use_tc_tiling_on_sc=False),
    scratch_types=[
        pltpu.VMEM_SHARED((_NP, _NF), jnp.float32),
        pltpu.VMEM_SHARED((_NP, _NF), jnp.float32),
        pltpu.VMEM((2, _CHB), jnp.int32),
        pltpu.VMEM((2, _CHB), jnp.int32),
        pltpu.VMEM((_CHB, _NF), jnp.float32),
        pltpu.VMEM((2, _CHB // 2, 2 * _NF), jnp.float32),
        pltpu.VMEM((_ZR, _NF), jnp.float32),
        pltpu.SemaphoreType.DMA,
        pltpu.SemaphoreType.DMA,
        pltpu.SemaphoreType.DMA,
        pltpu.SemaphoreType.DMA,
        pltpu.SemaphoreType.DMA,
        pltpu.SemaphoreType.DMA,
    ],
)
def _sc_msg(h_hbm, w_hbm, src_hbm, dst_hbm, out_hbm,
            acc, hsh, sv, dv, hs, wv, zb, ss0, ss1, sd0, sd1, sw0, sw1):
    c = lax.axis_index("c")
    s = lax.axis_index("s")
    wid = s * _NC + c
    z16 = jnp.zeros((_L,), jnp.float32)
    ss = (ss0, ss1)
    sd = (sd0, sd1)
    sw = (sw0, sw1)
    nk = _EPTH // _CHB

    def zrow(r, carry):
        for ci in range(_NF // _L):
            zb[r, pl.ds(ci * _L, _L)] = z16
        return carry

    lax.fori_loop(0, _ZR, zrow, 0)
    row_s = s * _RPS
    for k2 in range(_RPS // _ZR):
        pltpu.sync_copy(zb, acc.at[pl.ds(row_s + k2 * _ZR, _ZR)])
    pltpu.sync_copy(h_hbm.at[pl.ds(row_s, _RPS)], hsh.at[pl.ds(row_s, _RPS)])
    plsc.subcore_barrier()

    def start(slot, k):
        base = wid * _EPTH + k * _CHB
        pltpu.async_copy(src_hbm.at[pl.ds(base, _CHB)], sv.at[slot], ss[slot])
        pltpu.async_copy(dst_hbm.at[pl.ds(base, _CHB)], dv.at[slot], sd[slot])
        pltpu.async_copy(w_hbm.at[pl.ds(base // 2, _CHB // 2)], wv.at[slot],
                         sw[slot])

    def wait_idx(slot):
        pltpu.make_async_copy(src_hbm.at[pl.ds(0, _CHB)], sv.at[slot],
                              ss[slot]).wait()
        pltpu.make_async_copy(dst_hbm.at[pl.ds(0, _CHB)], dv.at[slot],
                              sd[slot]).wait()
        pltpu.make_async_copy(w_hbm.at[pl.ds(0, _CHB // 2)], wv.at[slot],
                              sw[slot]).wait()

    def process(slot):
        pltpu.sync_copy(hsh.at[sv.at[slot]], hs)

        def mul(j, carry2):
            for ci in range(_NF // _L):
                sl = pl.ds(ci * _L, _L)
                hs[2 * j, sl] = hs[2 * j, sl] * wv[slot, j, pl.ds(ci * _L, _L)]
                hs[2 * j + 1, sl] = (hs[2 * j + 1, sl]
                                     * wv[slot, j, pl.ds(_NF + ci * _L, _L)])
            return carry2

        lax.fori_loop(0, _CHB // 2, mul, 0)
        pltpu.sync_copy(hs, acc.at[dv.at[slot]], add=True)

    start(0, 0)

    def pair(kk, carry):
        k0 = 2 * kk
        start(1, k0 + 1)
        wait_idx(0)
        process(0)
        start(0, lax.rem(k0 + 2, nk))
        wait_idx(1)
        process(1)
        return carry

    lax.fori_loop(0, nk // 2, pair, 0)
    wait_idx(0)
    plsc.subcore_barrier()

    for k2 in range(_RPS // _ZR):
        row0 = row_s + k2 * _ZR
        pltpu.sync_copy(acc.at[pl.ds(row0, _ZR)], zb)
        pltpu.sync_copy(zb, out_hbm.at[c, pl.ds(row0, _ZR)])


def _h_body(x_ref, w_ref, o_ref):
    o_ref[...] = jnp.dot(x_ref[...], w_ref[...],
                         preferred_element_type=jnp.float32)


def _tc_h(x, w_cf1):
    return pl.pallas_call(
        _h_body,
        grid=(10,),
        in_specs=[
            pl.BlockSpec((_N // 10, _D), lambda i: (i, 0)),
            pl.BlockSpec((_D, _NF), lambda i: (0, 0)),
        ],
        out_specs=pl.BlockSpec((_N // 10, _NF), lambda i: (i, 0)),
        out_shape=jax.ShapeDtypeStruct((_N, _NF), jnp.float32),
    )(x, w_cf1)


_BE = 2048


def _make_filter_body(off):
    def _filter_body(d2_ref, w1t_ref, b1_ref, w2t_ref, b2_ref, o_ref):
        pid = pl.program_id(0)
        eid = (pid + off) * _BE + lax.broadcasted_iota(jnp.int32, (1, _BE), 1)
        valid = eid < _E
        d2 = jnp.where(valid, d2_ref[...].reshape(1, _BE), 0.0)
        dist = jnp.sqrt(d2 + 1e-12)
        cw = 0.5 * (jnp.cos(dist * (math.pi / _CUTOFF)) + 1.0)
        gi = lax.broadcasted_iota(jnp.int32, (_NG, _BE), 0)
        g = gi.astype(jnp.float32)
        delta = _CUTOFF / (_NG - 1)
        coeff = -0.5 / (delta * delta)
        diff = dist - g * delta
        ea = jnp.exp(coeff * diff * diff)
        t = jnp.dot(w1t_ref[...], ea, preferred_element_type=jnp.float32)
        t = _ssp(t + b1_ref[...])
        t = jnp.dot(w2t_ref[...], t, preferred_element_type=jnp.float32)
        t = t + b2_ref[...]
        t = t * jnp.where(valid, cw, 0.0)
        ii = (lax.broadcasted_iota(jnp.int32, (_NF, _NF), 0)
              == lax.broadcasted_iota(jnp.int32, (_NF, _NF), 1)
              ).astype(jnp.float32)
        dn = (((0,), (0,)), ((), ()))
        ta = lax.dot_general(t[:, : _BE // 2], ii, dn,
                             preferred_element_type=jnp.float32)
        tb = lax.dot_general(t[:, _BE // 2:], ii, dn,
                             preferred_element_type=jnp.float32)
        o_ref[...] = jnp.concatenate([ta, tb], axis=1)

    return _filter_body


def _make_tc_filter(off):
    body = _make_filter_body(off)

    def _filter(d2, w1t, b1, w2t, b2):
        return pl.pallas_call(
            body,
            grid=(_EH // _BE,),
            in_specs=[
                pl.BlockSpec((_BE,), lambda i: (i + off,)),
                pl.BlockSpec((_NF, _NG), lambda i: (0, 0)),
                pl.BlockSpec((_NF, 1), lambda i: (0, 0)),
                pl.BlockSpec((_NF, _NF), lambda i: (0, 0)),
                pl.BlockSpec((_NF, 1), lambda i: (0, 0)),
            ],
            out_specs=pl.BlockSpec((_BE // 2, 2 * _NF), lambda i: (i, 0)),
            out_shape=jax.ShapeDtypeStruct((_EH // 2, 2 * _NF), jnp.float32),
        )(d2, w1t, b1, w2t, b2)

    return _filter


_tc_filter_a = _make_tc_filter(0)
_tc_filter_b = _make_tc_filter(_EH // _BE)


def _tail_body(x_ref, p0_ref, p1_ref, p2_ref, p3_ref,
               wcf2_ref, bcf2_ref, wint_ref, bint_ref,
               wlin_ref, blin_ref, o_ref):
    agg = (p0_ref[0] + p1_ref[0]) + (p2_ref[0] + p3_ref[0])
    h2 = jnp.dot(agg, wcf2_ref[...], preferred_element_type=jnp.float32)
    h2 = _ssp(h2 + bcf2_ref[...])
    h2 = jnp.dot(h2, wint_ref[...], preferred_element_type=jnp.float32)
    h2 = h2 + bint_ref[...]
    y = jnp.dot(h2, wlin_ref[...], preferred_element_type=jnp.float32)
    y = jnp.maximum(y + blin_ref[...], 0.0)
    o_ref[...] = x_ref[...] + y


def _tc_tail(x, parts_a, parts_b, w_cf2, b_cf2, w_int, b_int, w_lin1, b_lin1):
    br = _N // 10
    return pl.pallas_call(
        _tail_body,
        grid=(10,),
        in_specs=[
            pl.BlockSpec((br, _D), lambda i: (i, 0)),
            pl.BlockSpec((1, br, _NF), lambda i: (0, i, 0)),
            pl.BlockSpec((1, br, _NF), lambda i: (1, i, 0)),
            pl.BlockSpec((1, br, _NF), lambda i: (0, i, 0)),
            pl.BlockSpec((1, br, _NF), lambda i: (1, i, 0)),
            pl.BlockSpec((_NF, _D), lambda i: (0, 0)),
            pl.BlockSpec((1, _D), lambda i: (0, 0)),
            pl.BlockSpec((_D, _D), lambda i: (0, 0)),
            pl.BlockSpec((1, _D), lambda i: (0, 0)),
            pl.BlockSpec((_D, _D), lambda i: (0, 0)),
            pl.BlockSpec((1, _D), lambda i: (0, 0)),
        ],
        out_specs=pl.BlockSpec((br, _D), lambda i: (i, 0)),
        out_shape=jax.ShapeDtypeStruct((_N, _D), jnp.float32),
    )(x, parts_a, parts_a, parts_b, parts_b,
      w_cf2, b_cf2, w_int, b_int, w_lin1, b_lin1)


def kernel(x, pos, edge_index, W_mlp1, b_mlp1, W_mlp2, b_mlp2,
           W_cf1, W_cf2, b_cf2, W_int, b_int, W_lin1, b_lin1):
    src = edge_index[0]
    dst = edge_index[1]
    pad = _E_PAD - _E
    srcp = jnp.pad(src, (0, pad))
    dstp = jnp.pad(dst, (0, pad))
    eip = jnp.stack([srcp, dstp])
    posx = pos[:, 0]
    posy = pos[:, 1]
    posz = pos[:, 2]

    d2 = _sc_dist(posx, posy, posz, eip)
    h = _tc_h(x, W_cf1)

    w1t = W_mlp1.T
    b1c = b_mlp1.reshape(_NF, 1)
    w2t = W_mlp2.T
    b2c = b_mlp2.reshape(_NF, 1)
    w_a = _tc_filter_a(d2, w1t, b1c, w2t, b2c)
    w_b = _tc_filter_b(d2, w1t, b1c, w2t, b2c)

    def _ilv(v):
        return v.reshape(-1, 2, _BE // 2).transpose(0, 2, 1).reshape(-1)

    hp = jnp.pad(h, ((0, _NP - _N), (0, 0)))
    parts_a = _sc_msg(hp, w_a, _ilv(srcp[:_EH]), _ilv(dstp[:_EH]))
    parts_b = _sc_msg(hp, w_b, _ilv(srcp[_EH:]), _ilv(dstp[_EH:]))

    return _tc_tail(x, parts_a, parts_b, W_cf2, b_cf2.reshape(1, _D),
                    W_int, b_int.reshape(1, _D), W_lin1, b_lin1.reshape(1, _D))


# speedup vs baseline: 1.5801x; 1.0645x over previous
import functools
import math

import jax
import jax.numpy as jnp
from jax import lax
from jax.experimental import pallas as pl
from jax.experimental.pallas import tpu as pltpu
from jax.experimental.pallas import tpu_sc as plsc

_N = 10000
_E = 320000
_D = 128
_NG = 50
_NF = 64
_CUTOFF = 10.0
_LN2 = math.log(2.0)

_NC, _NS, _L = 2, 16, 16
_NW = _NC * _NS
_E_PAD = 327680
_EPT = _E_PAD // _NW

_EHA = _E_PAD // 4
_EHB = _E_PAD - _EHA
_EPTA = _EHA // _NW
_EPTB = _EHB // _NW

_CHA = 512
_CHB = 128
_NP = 10240
_RPS = _NP // _NS
_ZR = 128

_MESH = dict(core_axis_name="c", subcore_axis_name="s")


def _ssp(v):
    return jnp.maximum(v, 0.0) + jnp.log(1.0 + jnp.exp(-jnp.abs(v))) - _LN2


def _make_sc_dist(ev):
    @functools.partial(
        pl.kernel,
        out_type=jax.ShapeDtypeStruct((_E_PAD,), jnp.float32),
        mesh=plsc.VectorSubcoreMesh(**_MESH),
        compiler_params=pltpu.CompilerParams(needs_layout_passes=False, use_tc_tiling_on_sc=False),
        scratch_types=[
            pltpu.VMEM((_N,), jnp.float32),
            pltpu.VMEM((_N,), jnp.float32),
            pltpu.VMEM((_N,), jnp.float32),
            pltpu.VMEM((_CHA,), jnp.int32),
            pltpu.VMEM((_CHA,), jnp.int32),
            pltpu.VMEM((_CHA,), jnp.float32),
        ],
    )
    def _dist(px_hbm, py_hbm, pz_hbm, ei_hbm, d2_hbm,
              px, py, pz, sv, dv, ov):
        c = lax.axis_index("c")
        s = lax.axis_index("s")
        wid = s * _NC + c
        pltpu.sync_copy(px_hbm, px)
        pltpu.sync_copy(py_hbm, py)
        pltpu.sync_copy(pz_hbm, pz)
        nk = (jnp.maximum(jnp.minimum(ev, (wid + 1) * _EPT) - wid * _EPT, 0)
              // _CHA)

        def chunk(k, carry):
            base = wid * _EPT + k * _CHA
            pltpu.sync_copy(ei_hbm.at[0, pl.ds(base, _CHA)], sv)
            pltpu.sync_copy(ei_hbm.at[1, pl.ds(base, _CHA)], dv)

            def inner(i, carry2):
                s16 = sv[pl.ds(i * _L, _L)]
                d16 = dv[pl.ds(i * _L, _L)]
                dx = plsc.load_gather(px, [d16]) - plsc.load_gather(px, [s16])
                dy = plsc.load_gather(py, [d16]) - plsc.load_gather(py, [s16])
                dz = plsc.load_gather(pz, [d16]) - plsc.load_gather(pz, [s16])
                ov[pl.ds(i * _L, _L)] = dx * dx + dy * dy + dz * dz
                return carry2

            lax.fori_loop(0, _CHA // _L, inner, 0)
            pltpu.sync_copy(ov, d2_hbm.at[pl.ds(base, _CHA)])
            return carry

        lax.fori_loop(0, nk, chunk, 0)

    return _dist


_sc_dist = _make_sc_dist(_E)


def _sc_msg_body(ept, h_hbm, w_hbm, src_hbm, dst_hbm, out_hbm,
                 acc, hsh, sv, dv, hs, wv, zb, ss0, ss1, sd0, sd1, sw0, sw1):
    c = lax.axis_index("c")
    s = lax.axis_index("s")
    wid = s * _NC + c
    z16 = jnp.zeros((_L,), jnp.float32)
    ss = (ss0, ss1)
    sd = (sd0, sd1)
    sw = (sw0, sw1)
    nk = ept // _CHB

    def zrow(r, carry):
        for ci in range(_NF // _L):
            zb[r, pl.ds(ci * _L, _L)] = z16
        return carry

    lax.fori_loop(0, _ZR, zrow, 0)
    row_s = s * _RPS
    for k2 in range(_RPS // _ZR):
        pltpu.sync_copy(zb, acc.at[pl.ds(row_s + k2 * _ZR, _ZR)])
    pltpu.sync_copy(h_hbm.at[pl.ds(row_s, _RPS)], hsh.at[pl.ds(row_s, _RPS)])
    plsc.subcore_barrier()

    def start(slot, k):
        base = wid * ept + k * _CHB
        pltpu.async_copy(src_hbm.at[pl.ds(base, _CHB)], sv.at[slot], ss[slot])
        pltpu.async_copy(dst_hbm.at[pl.ds(base, _CHB)], dv.at[slot], sd[slot])
        pltpu.async_copy(w_hbm.at[pl.ds(base // 2, _CHB // 2)], wv.at[slot],
                         sw[slot])

    def wait_idx(slot):
        pltpu.make_async_copy(src_hbm.at[pl.ds(0, _CHB)], sv.at[slot],
                              ss[slot]).wait()
        pltpu.make_async_copy(dst_hbm.at[pl.ds(0, _CHB)], dv.at[slot],
                              sd[slot]).wait()
        pltpu.make_async_copy(w_hbm.at[pl.ds(0, _CHB // 2)], wv.at[slot],
                              sw[slot]).wait()

    def process(slot):
        pltpu.sync_copy(hsh.at[sv.at[slot]], hs)

        def mul(j, carry2):
            for ci in range(_NF // _L):
                sl = pl.ds(ci * _L, _L)
                hs[2 * j, sl] = hs[2 * j, sl] * wv[slot, j, pl.ds(ci * _L, _L)]
                hs[2 * j + 1, sl] = (hs[2 * j + 1, sl]
                                     * wv[slot, j, pl.ds(_NF + ci * _L, _L)])
            return carry2

        lax.fori_loop(0, _CHB // 2, mul, 0)
        pltpu.sync_copy(hs, acc.at[dv.at[slot]], add=True)

    start(0, 0)

    def pair(kk, carry):
        k0 = 2 * kk
        start(1, k0 + 1)
        wait_idx(0)
        process(0)
        start(0, lax.rem(k0 + 2, nk))
        wait_idx(1)
        process(1)
        return carry

    lax.fori_loop(0, nk // 2, pair, 0)
    wait_idx(0)
    plsc.subcore_barrier()

    for k2 in range(_RPS // _ZR):
        row0 = row_s + k2 * _ZR
        pltpu.sync_copy(acc.at[pl.ds(row0, _ZR)], zb)
        pltpu.sync_copy(zb, out_hbm.at[c, pl.ds(row0, _ZR)])


def _make_sc_msg(ept):
    def _msg(h_hbm, w_hbm, src_hbm, dst_hbm, out_hbm,
             acc, hsh, sv, dv, hs, wv, zb, ss0, ss1, sd0, sd1, sw0, sw1):
        _sc_msg_body(ept, h_hbm, w_hbm, src_hbm, dst_hbm, out_hbm,
                     acc, hsh, sv, dv, hs, wv, zb,
                     ss0, ss1, sd0, sd1, sw0, sw1)

    return pl.kernel(
        _msg,
        out_type=jax.ShapeDtypeStruct((_NC, _NP, _NF), jnp.float32),
        mesh=plsc.VectorSubcoreMesh(**_MESH),
        compiler_params=pltpu.CompilerParams(needs_layout_passes=False,
                                             use_tc_tiling_on_sc=False),
        scratch_types=[
            pltpu.VMEM_SHARED((_NP, _NF), jnp.float32),
            pltpu.VMEM_SHARED((_NP, _NF), jnp.float32),
            pltpu.VMEM((2, _CHB), jnp.int32),
            pltpu.VMEM((2, _CHB), jnp.int32),
            pltpu.VMEM((_CHB, _NF), jnp.float32),
            pltpu.VMEM((2, _CHB // 2, 2 * _NF), jnp.float32),
            pltpu.VMEM((_ZR, _NF), jnp.float32),
            pltpu.SemaphoreType.DMA,
            pltpu.SemaphoreType.DMA,
            pltpu.SemaphoreType.DMA,
            pltpu.SemaphoreType.DMA,
            pltpu.SemaphoreType.DMA,
            pltpu.SemaphoreType.DMA,
        ],
    )


_sc_msg_a = _make_sc_msg(_EPTA)
_sc_msg_b = _make_sc_msg(_EPTB)


def _h_body(x_ref, w_ref, o_ref):
    o_ref[...] = jnp.dot(x_ref[...], w_ref[...],
                         preferred_element_type=jnp.float32)


def _tc_h(x, w_cf1):
    return pl.pallas_call(
        _h_body,
        grid=(10,),
        in_specs=[
            pl.BlockSpec((_N // 10, _D), lambda i: (i, 0)),
            pl.BlockSpec((_D, _NF), lambda i: (0, 0)),
        ],
        out_specs=pl.BlockSpec((_N // 10, _NF), lambda i: (i, 0)),
        out_shape=jax.ShapeDtypeStruct((_N, _NF), jnp.float32),
    )(x, w_cf1)


_BE = 2048


def _make_filter_body(off):
    def _filter_body(d2_ref, w1t_ref, b1_ref, w2t_ref, b2_ref, o_ref):
        pid = pl.program_id(0)
        eid = (pid + off) * _BE + lax.broadcasted_iota(jnp.int32, (1, _BE), 1)
        valid = eid < _E
        d2 = jnp.where(valid, d2_ref[...].reshape(1, _BE), 0.0)
        dist = jnp.sqrt(d2 + 1e-12)
        cw = 0.5 * (jnp.cos(dist * (math.pi / _CUTOFF)) + 1.0)
        gi = lax.broadcasted_iota(jnp.int32, (_NG, _BE), 0)
        g = gi.astype(jnp.float32)
        delta = _CUTOFF / (_NG - 1)
        coeff = -0.5 / (delta * delta)
        diff = dist - g * delta
        ea = jnp.exp(coeff * diff * diff)
        t = jnp.dot(w1t_ref[...], ea, preferred_element_type=jnp.float32)
        t = _ssp(t + b1_ref[...])
        t = jnp.dot(w2t_ref[...], t, preferred_element_type=jnp.float32)
        t = t + b2_ref[...]
        t = t * jnp.where(valid, cw, 0.0)
        ii = (lax.broadcasted_iota(jnp.int32, (_NF, _NF), 0)
              == lax.broadcasted_iota(jnp.int32, (_NF, _NF), 1)
              ).astype(jnp.float32)
        dn = (((0,), (0,)), ((), ()))
        ta = lax.dot_general(t[:, : _BE // 2], ii, dn,
                             preferred_element_type=jnp.float32)
        tb = lax.dot_general(t[:, _BE // 2:], ii, dn,
                             preferred_element_type=jnp.float32)
        o_ref[...] = jnp.concatenate([ta, tb], axis=1)

    return _filter_body


def _make_tc_filter(off, ne):
    body = _make_filter_body(off)

    def _filter(d2, w1t, b1, w2t, b2):
        return pl.pallas_call(
            body,
            grid=(ne // _BE,),
            in_specs=[
                pl.BlockSpec((_BE,), lambda i: (i + off,)),
                pl.BlockSpec((_NF, _NG), lambda i: (0, 0)),
                pl.BlockSpec((_NF, 1), lambda i: (0, 0)),
                pl.BlockSpec((_NF, _NF), lambda i: (0, 0)),
                pl.BlockSpec((_NF, 1), lambda i: (0, 0)),
            ],
            out_specs=pl.BlockSpec((_BE // 2, 2 * _NF), lambda i: (i, 0)),
            out_shape=jax.ShapeDtypeStruct((ne // 2, 2 * _NF), jnp.float32),
        )(d2, w1t, b1, w2t, b2)

    return _filter


_tc_filter_a = _make_tc_filter(0, _EHA)
_tc_filter_b = _make_tc_filter(_EHA // _BE, _EHB)


def _tail_body(x_ref, p0_ref, p1_ref, p2_ref, p3_ref,
               wcf2_ref, bcf2_ref, wint_ref, bint_ref,
               wlin_ref, blin_ref, o_ref):
    agg = (p0_ref[0] + p1_ref[0]) + (p2_ref[0] + p3_ref[0])
    h2 = jnp.dot(agg, wcf2_ref[...], preferred_element_type=jnp.float32)
    h2 = _ssp(h2 + bcf2_ref[...])
    h2 = jnp.dot(h2, wint_ref[...], preferred_element_type=jnp.float32)
    h2 = h2 + bint_ref[...]
    y = jnp.dot(h2, wlin_ref[...], preferred_element_type=jnp.float32)
    y = jnp.maximum(y + blin_ref[...], 0.0)
    o_ref[...] = x_ref[...] + y


def _tc_tail(x, parts_a, parts_b, w_cf2, b_cf2, w_int, b_int, w_lin1, b_lin1):
    br = _N // 10
    return pl.pallas_call(
        _tail_body,
        grid=(10,),
        in_specs=[
            pl.BlockSpec((br, _D), lambda i: (i, 0)),
            pl.BlockSpec((1, br, _NF), lambda i: (0, i, 0)),
            pl.BlockSpec((1, br, _NF), lambda i: (1, i, 0)),
            pl.BlockSpec((1, br, _NF), lambda i: (0, i, 0)),
            pl.BlockSpec((1, br, _NF), lambda i: (1, i, 0)),
            pl.BlockSpec((_NF, _D), lambda i: (0, 0)),
            pl.BlockSpec((1, _D), lambda i: (0, 0)),
            pl.BlockSpec((_D, _D), lambda i: (0, 0)),
            pl.BlockSpec((1, _D), lambda i: (0, 0)),
            pl.BlockSpec((_D, _D), lambda i: (0, 0)),
            pl.BlockSpec((1, _D), lambda i: (0, 0)),
        ],
        out_specs=pl.BlockSpec((br, _D), lambda i: (i, 0)),
        out_shape=jax.ShapeDtypeStruct((_N, _D), jnp.float32),
    )(x, parts_a, parts_a, parts_b, parts_b,
      w_cf2, b_cf2, w_int, b_int, w_lin1, b_lin1)


def kernel(x, pos, edge_index, W_mlp1, b_mlp1, W_mlp2, b_mlp2,
           W_cf1, W_cf2, b_cf2, W_int, b_int, W_lin1, b_lin1):
    src = edge_index[0]
    dst = edge_index[1]
    pad = _E_PAD - _E
    srcp = jnp.pad(src, (0, pad))
    dstp = jnp.pad(dst, (0, pad))
    eip = jnp.stack([srcp, dstp])
    posx = pos[:, 0]
    posy = pos[:, 1]
    posz = pos[:, 2]

    d2 = _sc_dist(posx, posy, posz, eip)
    h = _tc_h(x, W_cf1)

    w1t = W_mlp1.T
    b1c = b_mlp1.reshape(_NF, 1)
    w2t = W_mlp2.T
    b2c = b_mlp2.reshape(_NF, 1)
    w_a = _tc_filter_a(d2, w1t, b1c, w2t, b2c)
    w_b = _tc_filter_b(d2, w1t, b1c, w2t, b2c)

    def _ilv(v):
        return v.reshape(-1, 2, _BE // 2).transpose(0, 2, 1).reshape(-1)

    hp = jnp.pad(h, ((0, _NP - _N), (0, 0)))
    parts_a = _sc_msg_a(hp, w_a, _ilv(srcp[:_EHA]), _ilv(dstp[:_EHA]))
    parts_b = _sc_msg_b(hp, w_b, _ilv(srcp[_EHA:]), _ilv(dstp[_EHA:]))

    return _tc_tail(x, parts_a, parts_b, W_cf2, b_cf2.reshape(1, _D),
                    W_int, b_int.reshape(1, _D), W_lin1, b_lin1.reshape(1, _D))
